# baseline (speedup 1.0000x reference)
import functools

import jax
import jax.numpy as jnp
from jax import lax
from jax.experimental import pallas as pl
from jax.experimental.pallas import tpu as pltpu
from jax.experimental.pallas import tpu_sc as plsc

B = 20
N_PER = 500
E_PER = 16000
E = B * E_PER
D = 128
K = 400

NC = 2
NS = 16
G_PER_SC = B // NC
DW = 32
NLOC = 512
MAXSPAN = 1536


def _lane_bcast(vec16, li):
    idx = jnp.full((16, 1), li, jnp.int32)
    return lax.gather(
        vec16, idx,
        dimension_numbers=lax.GatherDimensionNumbers(
            offset_dims=(), collapsed_slice_dims=(0,), start_index_map=(0,)),
        slice_sizes=(1,), mode=lax.GatherScatterMode.PROMISE_IN_BOUNDS)


def _sc_segsum_body(zero_self, vals_hbm, srcs_hbm, dsts_hbm, order_hbm, w_hbm,
                    q0_hbm, q1_hbm, out_hbm,
                    vals_v, src_v, dst_v, ord2d, w2d, wflat_v,
                    q0_v, q1_v, out_v, sem):
    c = lax.axis_index("c")
    s = lax.axis_index("s")
    iota16 = lax.iota(jnp.int32, 16)

    def graph_body(gi, _):
        g = c * G_PER_SC + gi
        pltpu.sync_copy(q0_hbm.at[g, pl.ds(s * DW, DW)], q0_v)
        pltpu.sync_copy(q1_hbm.at[g, pl.ds(s * DW, DW)], q1_v)
        q0c = [q0_v[pl.ds(0, 16)], q0_v[pl.ds(16, 16)]]
        q1c = [q1_v[pl.ds(0, 16)], q1_v[pl.ds(16, 16)]]
        p0 = q0c[0][0]
        p0a = (p0 // 8) * 8
        ga = (g * N_PER // 8) * 8
        vskew = g * N_PER - ga
        pltpu.sync_copy(vals_hbm.at[pl.ds(ga * D, 512 * D)], vals_v)
        pltpu.sync_copy(srcs_hbm.at[pl.ds(p0a, MAXSPAN)], src_v)
        if zero_self:
            pltpu.sync_copy(dsts_hbm.at[pl.ds(p0a, MAXSPAN)], dst_v)
        for j in range(MAXSPAN // 128):
            pltpu.sync_copy(order_hbm.at[pl.ds(p0a + j * 128, 128)],
                            ord2d.at[j])
        for j in range(MAXSPAN // 128):
            pltpu.async_copy(w_hbm.at[ord2d.at[j]], w2d.at[j], sem).wait()

        def wchunk(k, _):
            row = k // 8
            col = (k % 8) * 16
            wc = w2d[row, pl.ds(col, 16)]
            if zero_self:
                sc_ = src_v[pl.ds(k * 16, 16)]
                dc_ = dst_v[pl.ds(k * 16, 16)]
                wc = jnp.where(sc_ == dc_, 0.0, wc)
            wflat_v[pl.ds(k * 16, 16)] = wc
            return 0
        lax.fori_loop(0, MAXSPAN // 16, wchunk, 0, unroll=False)

        gbase = g * N_PER

        for l in range(DW):
            pe0 = q0c[l // 16][l % 16]
            pe1 = q1c[l // 16][l % 16]

            def edge_body(e, acc):
                le = e - p0a
                ci = (le // 16) * 16
                li = le - ci
                svec = _lane_bcast(src_v[pl.ds(ci, 16)], li) - gbase + vskew
                wvec = _lane_bcast(wflat_v[pl.ds(ci, 16)], li)
                rbase = svec * D
                return tuple(
                    acc[k]
                    + plsc.load_gather(vals_v, [rbase + (k * 16) + iota16])
                    * wvec
                    for k in range(8))

            z = jnp.zeros((16,), jnp.float32)
            acc = lax.fori_loop(pe0, pe1, edge_body, (z,) * 8)
            for k in range(8):
                out_v[l, pl.ds(k * 16, 16)] = acc[k]

        pltpu.sync_copy(out_v, out_hbm.at[pl.ds(g * NLOC + s * DW, DW)])
        return 0

    lax.fori_loop(0, G_PER_SC, graph_body, 0, unroll=False)


def _sc_segsum(vals, srcs, dsts, order, w, q0, q1, zero_self):
    mesh = plsc.VectorSubcoreMesh(core_axis_name="c", subcore_axis_name="s")
    kern = functools.partial(
        pl.kernel,
        out_type=jax.ShapeDtypeStruct((B * NLOC, D), jnp.float32),
        mesh=mesh,
        scratch_types=[
            pltpu.VMEM((512 * D,), jnp.float32),
            pltpu.VMEM((MAXSPAN,), jnp.int32),
            pltpu.VMEM((MAXSPAN,), jnp.int32),
            pltpu.VMEM((MAXSPAN // 128, 128), jnp.int32),
            pltpu.VMEM((MAXSPAN // 128, 128), jnp.float32),
            pltpu.VMEM((MAXSPAN,), jnp.float32),
            pltpu.VMEM((DW,), jnp.int32),
            pltpu.VMEM((DW,), jnp.int32),
            pltpu.VMEM((DW, D), jnp.float32),
            pltpu.SemaphoreType.DMA,
        ],
        compiler_params=pltpu.CompilerParams(needs_layout_passes=False),
    )(functools.partial(_sc_segsum_body, zero_self))
    vals_p = jnp.concatenate([vals, jnp.zeros((16, D), vals.dtype)]).reshape(-1)
    return kern(vals_p, srcs, dsts, order, w, q0, q1)


def _tc1_body(f_ref, w_ref, sn_ref, hs_ref):
    h0 = jnp.dot(f_ref[...], w_ref[...], preferred_element_type=jnp.float32)
    hs_ref[...] = h0 * sn_ref[...]


def _tc2_body(conv_ref, dn_ref, b_ref, sn_ref, h_ref, hp_ref):
    h = jnp.maximum(conv_ref[...] * dn_ref[...] + b_ref[...], 0.0)
    h_ref[...] = h
    hp_ref[...] = h * sn_ref[...]


def _tc3_body(h_ref, praw_ref, dn_ref, pooled_ref, readout_ref):
    h = h_ref[0]
    prop = praw_ref[0] * dn_ref[0]
    diff = jnp.abs(h - prop)

    acc = diff[:, 0:8]
    for kk in range(1, 16):
        acc = acc + diff[:, 8 * kk:8 * kk + 8]
    t4 = acc[:, 0:4] + acc[:, 4:8]
    t2 = t4[:, 0:2] + t4[:, 2:4]
    score = t2[:, 0:1] + t2[:, 1:2]

    iota_r = lax.broadcasted_iota(jnp.int32, (N_PER, N_PER), 0)
    iota_c = lax.broadcasted_iota(jnp.int32, (N_PER, N_PER), 1)

    key = lax.bitcast_convert_type(score, jnp.int32)
    ones_col = jnp.ones((N_PER, 1), jnp.float32)

    def _byte(shift):
        piece = ((key >> shift) & 0xFF).astype(jnp.float32)
        row = lax.dot_general(ones_col, piece, (((1,), (1,)), ((), ())),
                              preferred_element_type=jnp.float32)
        col = jnp.broadcast_to(piece, (N_PER, N_PER))
        return col, row

    c3, r3 = _byte(24)
    c2, r2 = _byte(16)
    c1, r1 = _byte(8)
    c0, r0 = _byte(0)

    gt = ((c3 > r3)
          | ((c3 == r3) & ((c2 > r2)
          | ((c2 == r2) & ((c1 > r1)
          | ((c1 == r1) & (c0 > r0)))))))
    eq = (c3 == r3) & (c2 == r2) & (c1 == r1) & (c0 == r0)

    C = jnp.where(gt | (eq & (iota_r < iota_c)), 1.0, 0.0)
    rank_row = jnp.sum(C, axis=0, keepdims=True)

    M = jnp.where(iota_r.astype(jnp.float32)
                  == jnp.broadcast_to(rank_row, (N_PER, N_PER)), 1.0, 0.0)
    pooled_full = jnp.dot(M, h, preferred_element_type=jnp.float32,
                          precision=jax.lax.Precision.HIGHEST)
    pooled = pooled_full[:K]
    pooled_ref[0] = pooled

    avg = jnp.sum(pooled, axis=0, keepdims=True) * (1.0 / K)
    mx = jnp.max(pooled, axis=0, keepdims=True)
    readout_ref[0] = jnp.concatenate([avg, mx], axis=1)


def kernel(feature, e_feat, edge_index, num_nodes, W, b):
    src, dst = edge_index[0], edge_index[1]
    N = B * N_PER

    order = jnp.argsort(dst, stable=True).astype(jnp.int32)
    src_s = jnp.take(src, order)
    dst_s = jnp.take(dst, order)
    pad_i = jnp.zeros((MAXSPAN,), jnp.int32)
    src_sp = jnp.concatenate([src_s, pad_i])
    dst_sp = jnp.concatenate([dst_s, pad_i])
    order_p = jnp.concatenate([order, pad_i])

    lo = jnp.minimum(jnp.arange(NLOC), N_PER)
    hi = jnp.minimum(jnp.arange(NLOC) + 1, N_PER)
    bnd0 = (jnp.arange(B)[:, None] * N_PER + lo[None, :]).reshape(-1)
    bnd1 = (jnp.arange(B)[:, None] * N_PER + hi[None, :]).reshape(-1)
    q0 = jnp.searchsorted(dst_s, bnd0, side='left').astype(jnp.int32)
    q1 = jnp.searchsorted(dst_s, bnd1, side='left').astype(jnp.int32)
    q0 = q0.reshape(B, NLOC)
    q1 = q1.reshape(B, NLOC)

    out_deg = jnp.bincount(src, length=N).astype(jnp.float32)
    in_deg = jnp.bincount(dst, length=N).astype(jnp.float32)
    sn = jnp.power(jnp.clip(out_deg, 1.0, None), -0.5)[:, None]
    dn = jnp.power(jnp.clip(in_deg, 1.0, None), -0.5)[:, None]

    RB = 1000
    hs = pl.pallas_call(
        _tc1_body,
        grid=(N // RB,),
        in_specs=[
            pl.BlockSpec((RB, D), lambda i: (i, 0)),
            pl.BlockSpec((D, D), lambda i: (0, 0)),
            pl.BlockSpec((RB, 1), lambda i: (i, 0)),
        ],
        out_specs=pl.BlockSpec((RB, D), lambda i: (i, 0)),
        out_shape=jax.ShapeDtypeStruct((N, D), jnp.float32),
    )(feature, W, sn)

    conv_pad = _sc_segsum(hs, src_sp, dst_sp, order_p, e_feat, q0, q1,
                          zero_self=False)
    conv = conv_pad.reshape(B, NLOC, D)[:, :N_PER].reshape(N, D)

    h, hp = pl.pallas_call(
        _tc2_body,
        grid=(N // RB,),
        in_specs=[
            pl.BlockSpec((RB, D), lambda i: (i, 0)),
            pl.BlockSpec((RB, 1), lambda i: (i, 0)),
            pl.BlockSpec((1, D), lambda i: (0, 0)),
            pl.BlockSpec((RB, 1), lambda i: (i, 0)),
        ],
        out_specs=[
            pl.BlockSpec((RB, D), lambda i: (i, 0)),
            pl.BlockSpec((RB, D), lambda i: (i, 0)),
        ],
        out_shape=[
            jax.ShapeDtypeStruct((N, D), jnp.float32),
            jax.ShapeDtypeStruct((N, D), jnp.float32),
        ],
    )(conv, dn, b.reshape(1, D), sn)

    prop_pad = _sc_segsum(hp, src_sp, dst_sp, order_p, e_feat, q0, q1,
                          zero_self=True)
    prop_raw = prop_pad.reshape(B, NLOC, D)[:, :N_PER].reshape(N, D)

    h3 = h.reshape(B, N_PER, D)
    p3 = prop_raw.reshape(B, N_PER, D)
    dn3 = dn.reshape(B, N_PER, 1)
    pooled, readout = pl.pallas_call(
        _tc3_body,
        grid=(B,),
        in_specs=[
            pl.BlockSpec((1, N_PER, D), lambda g: (g, 0, 0)),
            pl.BlockSpec((1, N_PER, D), lambda g: (g, 0, 0)),
            pl.BlockSpec((1, N_PER, 1), lambda g: (g, 0, 0)),
        ],
        out_specs=[
            pl.BlockSpec((1, K, D), lambda g: (g, 0, 0)),
            pl.BlockSpec((1, 1, 256), lambda g: (g, 0, 0)),
        ],
        out_shape=[
            jax.ShapeDtypeStruct((B, K, D), jnp.float32),
            jax.ShapeDtypeStruct((B, 1, 256), jnp.float32),
        ],
    )(h3, p3, dn3)
    return pooled.reshape(B * K, D), readout.reshape(B, 256)
